# baseline (speedup 1.0000x reference)
import jax
import jax.numpy as jnp
from jax.experimental import pallas as pl
from jax.experimental.pallas import tpu as pltpu

_MAGIC = 8388608.0
_MAGIC_BITS = 0x4B000000


def _body(x_ref, rot_ref, eig_ref, mean_ref, cent_ref,
          xhat_ref, idx_ref, rt_scr, r16_scr, b_scr, k_scr):
    d = rot_ref.shape[0]
    k = cent_ref.shape[0]

    @pl.when(pl.program_id(0) == 0)
    def _init():
        rt_scr[...] = rot_ref[...].T
        r16_scr[...] = rot_ref[...] * 16.0
        eig = jnp.maximum(eig_ref[...], 1e-12)
        inv_ws = 1.0 / jnp.sqrt((1.0 / d) / eig)
        mrow = jnp.dot(mean_ref[...], rot_ref[...].T,
                       preferred_element_type=jnp.float32)
        for j in range(k):
            k_scr[j:j + 1, :] = (1.0 * j
                                 + (cent_ref[j] * inv_ws + mrow) * (1.0 / 16.0))
            jn = min(j + 1, k - 1)
            b_scr[j:j + 1, :] = (0.5 * (cent_ref[j] + cent_ref[jn])) * inv_ws

    yr = jnp.dot(x_ref[...] - mean_ref[...], rt_scr[...],
                 preferred_element_type=jnp.float32)

    q = k // 4
    wq = []
    for a in range(4):
        wa = k_scr[4 * a:4 * a + 1, :]
        for j in range(4 * a, 4 * a + q - 1):
            wa = jnp.where(yr > b_scr[j:j + 1, :], k_scr[j + 1:j + 2, :], wa)
        wq.append(wa)
    w_low = jnp.where(yr > b_scr[q - 1:q, :], wq[1], wq[0])
    w_high = jnp.where(yr > b_scr[3 * q - 1:3 * q, :], wq[3], wq[2])
    w = jnp.where(yr > b_scr[2 * q - 1:2 * q, :], w_high, w_low)

    qm = w + _MAGIC
    idx_i32 = (jax.lax.bitcast_convert_type(qm, jnp.int32)
               - jnp.int32(_MAGIC_BITS))
    qf = idx_i32.astype(jnp.float32)
    yh = w - qf

    xhat_ref[...] = jnp.dot(yh, r16_scr[...],
                            preferred_element_type=jnp.float32)
    idx_ref[...] = idx_i32


def kernel(x, rotation, eigenvalues, mean, centroids):
    n, d = x.shape
    tile = 4096
    grid = (n // tile,)
    eig2 = eigenvalues.reshape(1, d)
    mean2 = mean.reshape(1, d)

    x_hat, idx = pl.pallas_call(
        _body,
        grid=grid,
        in_specs=[
            pl.BlockSpec((tile, d), lambda i: (i, 0)),
            pl.BlockSpec((d, d), lambda i: (0, 0)),
            pl.BlockSpec((1, d), lambda i: (0, 0)),
            pl.BlockSpec((1, d), lambda i: (0, 0)),
            pl.BlockSpec(memory_space=pltpu.SMEM),
        ],
        out_specs=[
            pl.BlockSpec((tile, d), lambda i: (i, 0)),
            pl.BlockSpec((tile, d), lambda i: (i, 0)),
        ],
        out_shape=[
            jax.ShapeDtypeStruct((n, d), jnp.float32),
            jax.ShapeDtypeStruct((n, d), jnp.int32),
        ],
        scratch_shapes=[
            pltpu.VMEM((128, 128), jnp.float32),
            pltpu.VMEM((128, 128), jnp.float32),
            pltpu.VMEM((16, 128), jnp.float32),
            pltpu.VMEM((16, 128), jnp.float32),
        ],
    )(x, rotation, eig2, mean2, centroids)
    return x_hat, idx


# speedup vs baseline: 1.0118x; 1.0118x over previous
import jax
import jax.numpy as jnp
from jax.experimental import pallas as pl
from jax.experimental.pallas import tpu as pltpu

_MAGIC = 8388608.0
_MAGIC_BITS = 0x4B000000


def _body(x_ref, rot_ref, eig_ref, mean_ref, cent_ref,
          xhat_ref, idx_ref, rt_scr, r16_scr, b_scr, k_scr):
    d = rot_ref.shape[0]
    k = cent_ref.shape[0]

    @pl.when(pl.program_id(0) == 0)
    def _init():
        rt_scr[...] = rot_ref[...].T
        r16_scr[...] = rot_ref[...] * 16.0
        eig = jnp.maximum(eig_ref[...], 1e-12)
        inv_ws = 1.0 / jnp.sqrt((1.0 / d) / eig)
        mrow = jnp.dot(mean_ref[...], rot_ref[...].T,
                       preferred_element_type=jnp.float32)
        for j in range(k):
            k_scr[j:j + 1, :] = (1.0 * j
                                 + (cent_ref[j] * inv_ws + mrow) * (1.0 / 16.0))
            jn = min(j + 1, k - 1)
            b_scr[j:j + 1, :] = (0.5 * (cent_ref[j] + cent_ref[jn])) * inv_ws

    yr = jnp.dot(x_ref[...] - mean_ref[...], rt_scr[...],
                 preferred_element_type=jnp.float32)

    half = k // 2
    w_lo = k_scr[0:1, :]
    w_hi = k_scr[half:half + 1, :]
    for j in range(half - 1):
        w_lo = jnp.where(yr > b_scr[j:j + 1, :], k_scr[j + 1:j + 2, :], w_lo)
        jh = half + j
        w_hi = jnp.where(yr > b_scr[jh:jh + 1, :], k_scr[jh + 1:jh + 2, :], w_hi)
    w = jnp.where(yr > b_scr[half - 1:half, :], w_hi, w_lo)

    qm = w + _MAGIC
    idx_i32 = (jax.lax.bitcast_convert_type(qm, jnp.int32)
               - jnp.int32(_MAGIC_BITS))
    qf = idx_i32.astype(jnp.float32)
    yh = w - qf

    xhat_ref[...] = jnp.dot(yh, r16_scr[...],
                            preferred_element_type=jnp.float32)
    idx_ref[...] = idx_i32


def kernel(x, rotation, eigenvalues, mean, centroids):
    n, d = x.shape
    tile = 4096
    grid = (n // tile,)
    eig2 = eigenvalues.reshape(1, d)
    mean2 = mean.reshape(1, d)

    x_hat, idx = pl.pallas_call(
        _body,
        grid=grid,
        in_specs=[
            pl.BlockSpec((tile, d), lambda i: (i, 0)),
            pl.BlockSpec((d, d), lambda i: (0, 0)),
            pl.BlockSpec((1, d), lambda i: (0, 0)),
            pl.BlockSpec((1, d), lambda i: (0, 0)),
            pl.BlockSpec(memory_space=pltpu.SMEM),
        ],
        out_specs=[
            pl.BlockSpec((tile, d), lambda i: (i, 0)),
            pl.BlockSpec((tile, d), lambda i: (i, 0)),
        ],
        out_shape=[
            jax.ShapeDtypeStruct((n, d), jnp.float32),
            jax.ShapeDtypeStruct((n, d), jnp.int32),
        ],
        scratch_shapes=[
            pltpu.VMEM((128, 128), jnp.float32),
            pltpu.VMEM((128, 128), jnp.float32),
            pltpu.VMEM((16, 128), jnp.float32),
            pltpu.VMEM((16, 128), jnp.float32),
        ],
    )(x, rotation, eig2, mean2, centroids)
    return x_hat, idx
